# Initial kernel scaffold; baseline (speedup 1.0000x reference)
import jax
import jax.numpy as jnp
from jax.experimental import pallas as pl


def kernel(y, x, local_table, given_table, space_table, t2v_w, t2v_b, vt_w, vt_b):
    raise NotImplementedError("write your pallas kernel here")


# speedup vs baseline: 9.2141x; 9.2141x over previous
import jax
import jax.numpy as jnp
from jax import lax
from jax.experimental import pallas as pl
from jax.experimental.pallas import tpu as pltpu


def _body(y_ref, x_ref, lt_ref, gt_ref, st_ref, ew_ref, bf_ref, vtw_ref,
          wy_ref, vtb_ref, val_ref, sp_ref, comb_ref):
    v = pl.program_id(1)
    L, D = comb_ref.shape
    F = ew_ref.shape[1]
    K = F // x_ref.shape[2]

    @pl.when(v == 0)
    def _compute_combined():
        xb = x_ref[0]
        xb = jnp.where(jnp.isnan(xb), 0.0, xb)
        xa = jnp.dot(xb, ew_ref[...],
                     preferred_element_type=jnp.float32) + bf_ref[...]
        ksel = (lax.broadcasted_iota(jnp.int32, (1, F), 1) % K) > 0
        feats = jnp.where(ksel, jnp.sin(xa), xa)
        tp = jnp.dot(feats, vtw_ref[...],
                     preferred_element_type=jnp.float32)
        comb_ref[...] = (tp + lt_ref[...] + vtb_ref[...] + gt_ref[1:2, :])

    y2d = y_ref[0]
    nan2d = jnp.isnan(y2d)
    ycl = jnp.where(nan2d, 0.0, y2d)
    sel = (lax.broadcasted_iota(jnp.int32, (1, y2d.shape[1]), 1)
           == v).astype(jnp.float32)
    yc = jnp.sum(ycl * sel, axis=1, keepdims=True)
    nanf = jnp.sum(nan2d.astype(jnp.float32) * sel, axis=1, keepdims=True)
    delta = gt_ref[0:1, :] - gt_ref[1:2, :]
    val_ref[0] = comb_ref[...] + yc * wy_ref[...] + nanf * delta
    sp_ref[0] = jnp.broadcast_to(st_ref[0], (L, D))


def kernel(y, x, local_table, given_table, space_table, t2v_w, t2v_b,
           vt_w, vt_b):
    B, L, DY = y.shape
    DX = x.shape[-1]
    D = local_table.shape[-1]
    K = t2v_w.shape[-1]
    F = DX * K
    T = DY * L

    wf = t2v_w.reshape(1, F)
    bf = t2v_b.reshape(1, F)
    eye = (jnp.arange(F)[None, :] // K == jnp.arange(DX)[:, None])
    ew = eye.astype(jnp.float32) * wf
    vtw_t = vt_w[:, :F].T
    wy_row = vt_w[:, F:F + 1].T
    vtb_row = vt_b.reshape(1, D)

    grid = (B, DY)
    val, sp = pl.pallas_call(
        _body,
        grid=grid,
        in_specs=[
            pl.BlockSpec((1, L, DY), lambda b, v: (b, 0, 0)),
            pl.BlockSpec((1, L, DX), lambda b, v: (b, 0, 0)),
            pl.BlockSpec((L, D), lambda b, v: (0, 0)),
            pl.BlockSpec((2, D), lambda b, v: (0, 0)),
            pl.BlockSpec((1, 1, D), lambda b, v: (v, 0, 0)),
            pl.BlockSpec((DX, F), lambda b, v: (0, 0)),
            pl.BlockSpec((1, F), lambda b, v: (0, 0)),
            pl.BlockSpec((F, D), lambda b, v: (0, 0)),
            pl.BlockSpec((1, D), lambda b, v: (0, 0)),
            pl.BlockSpec((1, D), lambda b, v: (0, 0)),
        ],
        out_specs=[
            pl.BlockSpec((1, L, D), lambda b, v: (b, v, 0)),
            pl.BlockSpec((1, L, D), lambda b, v: (b, v, 0)),
        ],
        out_shape=[
            jax.ShapeDtypeStruct((B, T, D), jnp.float32),
            jax.ShapeDtypeStruct((B, T, D), jnp.float32),
        ],
        scratch_shapes=[pltpu.VMEM((L, D), jnp.float32)],
        compiler_params=pltpu.CompilerParams(
            dimension_semantics=("arbitrary", "arbitrary")),
    )(y, x, local_table, given_table, space_table.reshape(DY, 1, D), ew, bf,
      vtw_t, wy_row, vtb_row)

    var_idx = jnp.broadcast_to(
        jnp.repeat(jnp.arange(DY, dtype=jnp.int32), L), (B, T))
    mask = jnp.ones((B, T), jnp.int32)
    return (val, sp, var_idx, mask)
